# baseline (speedup 1.0000x reference)
import functools

import jax
import jax.numpy as jnp
from jax import lax
from jax.experimental import pallas as pl
from jax.experimental.pallas import tpu as pltpu
from jax.experimental.pallas import tpu_sc as plsc

D = 64
L = 16
MAX_NORM = 1.0
EPS = 1e-7
NC = 2
NS = 16
NW = NC * NS
BLK = 128
NBUF = 4


@functools.lru_cache(maxsize=None)
def _build(NB, T):
    assert NB == NW * BLK and T % NBUF == 0 and D % L == 0

    mesh = plsc.VectorSubcoreMesh(core_axis_name="c", subcore_axis_name="s")

    @functools.partial(
        pl.kernel,
        mesh=mesh,
        compiler_params=pltpu.CompilerParams(use_tc_tiling_on_sc=False,
                                             needs_layout_passes=False),
        out_type=jax.ShapeDtypeStruct((T, D // 8, NB // BLK, 8, BLK),
                                      jnp.float32),
        scratch_types=[
            pltpu.VMEM((T, BLK), jnp.int32),
            pltpu.VMEM((NBUF, BLK, D), jnp.float32),
            pltpu.VMEM((NBUF, D, BLK + 1), jnp.float32),
            pltpu.SemaphoreType.DMA((NBUF,)),
            pltpu.SemaphoreType.DMA((NBUF,)),
        ],
    )
    def k(table_hbm, idxt_hbm, out_hbm, idx_v, gbuf, obuf, gsem, osem):
        wid = lax.axis_index("s") * NC + lax.axis_index("c")
        pltpu.sync_copy(idxt_hbm.at[:, pl.ds(wid * BLK, BLK)], idx_v)

        lanes = lax.iota(jnp.int32, L)
        perms = [lanes ^ p for p in (1, 2, 4, 8)]
        kvecs = [lanes + m * L for m in range(D // L)]

        def gather(c, b):
            return pltpu.make_async_copy(
                table_hbm.at[idx_v.at[c]], gbuf.at[b], gsem.at[b])

        def putbacks(c, b):
            return [
                pltpu.make_async_copy(
                    obuf.at[b, pl.ds(g * 8, 8), pl.ds(0, BLK)],
                    out_hbm.at[c, g, wid], osem.at[b])
                for g in range(D // 8)
            ]

        for b in range(NBUF):
            gather(b, b).start()

        @pl.loop(0, T, step=NBUF)
        def _(c0):
            for b in range(NBUF):
                c = c0 + b
                gather(c, b).wait()

                @pl.when(c0 > 0)
                def _():
                    for p in putbacks(c - NBUF, b):
                        p.wait()

                gb = gbuf.at[b]
                ob = obuf.at[b]

                @plsc.parallel_loop(0, BLK, unroll=8)
                def _(r):
                    a = [gb[r, pl.ds(m * L, L)] for m in range(D // L)]
                    n = jnp.abs(a[0]) + jnp.abs(a[1])
                    for m in range(2, D // L):
                        n = n + jnp.abs(a[m])
                    for p in perms:
                        n = n + n.at[p].get(mode="promise_in_bounds")
                    s = jnp.where(n > MAX_NORM, MAX_NORM / (n + EPS),
                                  jnp.float32(1.0))
                    rvec = jnp.full((L,), r, dtype=jnp.int32)
                    for m in range(D // L):
                        plsc.store_scatter(ob, [kvecs[m], rvec], a[m] * s)

                @pl.when(c + NBUF < T)
                def _():
                    gather(c + NBUF, b).start()

                for p in putbacks(c, b):
                    p.start()

        for b in range(NBUF):
            for p in putbacks(T - NBUF + b, b):
                p.wait()

    return k


def kernel(x, table):
    NB, T = x.shape
    out5 = _build(NB, T)(table, x.T)
    return out5.transpose(2, 4, 0, 1, 3).reshape(NB, T, D)


# speedup vs baseline: 1.1173x; 1.1173x over previous
import functools

import jax
import jax.numpy as jnp
from jax import lax
from jax.experimental import pallas as pl
from jax.experimental.pallas import tpu as pltpu
from jax.experimental.pallas import tpu_sc as plsc

D = 64
L = 16
MAX_NORM = 1.0
EPS = 1e-7
NC = 2
NS = 16
NW = NC * NS
BLK = 128
NBUF = 2


@functools.lru_cache(maxsize=None)
def _build(NB, T):
    assert NB == NW * BLK and T % NBUF == 0 and D % L == 0

    mesh = plsc.VectorSubcoreMesh(core_axis_name="c", subcore_axis_name="s")

    @functools.partial(
        pl.kernel,
        mesh=mesh,
        compiler_params=pltpu.CompilerParams(use_tc_tiling_on_sc=False,
                                             needs_layout_passes=False),
        out_type=jax.ShapeDtypeStruct((T, D // 8, NB // BLK, 8, BLK),
                                      jnp.float32),
        scratch_types=[
            pltpu.VMEM((T, BLK), jnp.int32),
            pltpu.VMEM((NBUF, BLK, D), jnp.float32),
            pltpu.VMEM((NBUF, D, BLK + 1), jnp.float32),
            pltpu.SemaphoreType.DMA((NBUF,)),
            pltpu.SemaphoreType.DMA((NBUF,)),
        ],
    )
    def k(table_hbm, idxt_hbm, out_hbm, idx_v, gbuf, obuf, gsem, osem):
        wid = lax.axis_index("s") * NC + lax.axis_index("c")
        pltpu.sync_copy(idxt_hbm.at[:, pl.ds(wid * BLK, BLK)], idx_v)

        lanes = lax.iota(jnp.int32, L)
        perms = [lanes ^ p for p in (1, 2, 4, 8)]
        kvecs = [lanes + m * L for m in range(D // L)]

        def gather(c, b):
            return pltpu.make_async_copy(
                table_hbm.at[idx_v.at[c]], gbuf.at[b], gsem.at[b])

        def putbacks(c, b):
            return [
                pltpu.make_async_copy(
                    obuf.at[b, pl.ds(g * 8, 8), pl.ds(0, BLK)],
                    out_hbm.at[c, g, wid], osem.at[b])
                for g in range(D // 8)
            ]

        for b in range(NBUF):
            gather(b, b).start()

        @pl.loop(0, T, step=NBUF)
        def _(c0):
            for b in range(NBUF):
                c = c0 + b
                gather(c, b).wait()

                @pl.when(c0 > 0)
                def _():
                    for p in putbacks(c - NBUF, b):
                        p.wait()

                gb = gbuf.at[b]
                ob = obuf.at[b]

                @plsc.parallel_loop(0, BLK, unroll=8)
                def _(r):
                    a = [gb[r, pl.ds(m * L, L)] for m in range(D // L)]
                    n = jnp.abs(a[0]) + jnp.abs(a[1])
                    for m in range(2, D // L):
                        n = n + jnp.abs(a[m])
                    for p in perms:
                        n = n + n.at[p].get(mode="promise_in_bounds")
                    s = jnp.where(n > MAX_NORM, MAX_NORM / (n + EPS),
                                  jnp.float32(1.0))
                    rvec = jnp.full((L,), r, dtype=jnp.int32)
                    for m in range(D // L):
                        plsc.store_scatter(ob, [kvecs[m], rvec], a[m] * s)

                @pl.when(c + NBUF < T)
                def _():
                    gather(c + NBUF, b).start()

                for p in putbacks(c, b):
                    p.start()

        for b in range(NBUF):
            for p in putbacks(T - NBUF + b, b):
                p.wait()

    return k


def kernel(x, table):
    NB, T = x.shape
    out5 = _build(NB, T)(table, x.T)
    return out5.transpose(2, 4, 0, 1, 3).reshape(NB, T, D)
